# baseline (speedup 1.0000x reference)
import jax
import jax.numpy as jnp
from jax.experimental import pallas as pl
from jax.experimental.pallas import tpu as pltpu

_B, _N, _D, _H, _C = 16, 1024, 128, 128, 10
_HS = _N // 2
_CP = 128


def _fused_body(adj_ref, x_ref, wr_ref, wn_ref, b_ref, wc_ref, bc_ref,
                out_ref, agg_ref):
    k = pl.program_id(1)
    A = (adj_ref[0] > 0.5).astype(jnp.bfloat16)
    xh = x_ref[0]

    xs = x_ref[0, pl.ds(k * _HS, _HS), :]
    part = jax.lax.dot_general(
        A, xs,
        dimension_numbers=(((0,), (0,)), ((), ())),
        preferred_element_type=jnp.float32)

    @pl.when(k == 0)
    def _():
        agg_ref[...] = part

    @pl.when(k == 1)
    def _():
        agg = agg_ref[...] + part
        h = jax.lax.dot_general(
            xh, wr_ref[...],
            dimension_numbers=(((1,), (0,)), ((), ())),
            preferred_element_type=jnp.float32)
        h = h + jax.lax.dot_general(
            agg.astype(jnp.bfloat16), wn_ref[...],
            dimension_numbers=(((1,), (0,)), ((), ())),
            preferred_element_type=jnp.float32)
        h = jnp.maximum(h + b_ref[...], 0.0)
        pooled = jnp.sum(h, axis=0, keepdims=True) * (1.0 / _N)
        out_ref[0] = jnp.dot(pooled, wc_ref[...],
                             preferred_element_type=jnp.float32) + bc_ref[...]


def kernel(x, adj, W_root, W_nbr, b, W_cls, b_cls):
    xh = x.astype(jnp.bfloat16)
    wrh = W_root.astype(jnp.bfloat16)
    wnh = W_nbr.astype(jnp.bfloat16)
    b2 = b.reshape(1, _H)
    wc = jnp.zeros((_H, _CP), jnp.float32).at[:, :_C].set(W_cls)
    bc = jnp.zeros((1, _CP), jnp.float32).at[0, :_C].set(b_cls)

    out = pl.pallas_call(
        _fused_body,
        grid=(_B, 2),
        in_specs=[
            pl.BlockSpec((1, _HS, _N), lambda i, k: (i, k, 0)),
            pl.BlockSpec((1, _N, _D), lambda i, k: (i, 0, 0)),
            pl.BlockSpec((_D, _H), lambda i, k: (0, 0)),
            pl.BlockSpec((_D, _H), lambda i, k: (0, 0)),
            pl.BlockSpec((1, _H), lambda i, k: (0, 0)),
            pl.BlockSpec((_H, _CP), lambda i, k: (0, 0)),
            pl.BlockSpec((1, _CP), lambda i, k: (0, 0)),
        ],
        out_specs=pl.BlockSpec((1, 1, _CP), lambda i, k: (i, 0, 0)),
        out_shape=jax.ShapeDtypeStruct((_B, 1, _CP), jnp.float32),
        scratch_shapes=[pltpu.VMEM((_N, _D), jnp.float32)],
        compiler_params=pltpu.CompilerParams(
            dimension_semantics=("arbitrary", "arbitrary")),
    )(adj, xh, wrh, wnh, b2, wc, bc)
    return out[:, 0, :_C]


# speedup vs baseline: 1.5953x; 1.5953x over previous
import jax
import jax.numpy as jnp
from jax.experimental import pallas as pl
from jax.experimental.pallas import tpu as pltpu

_B, _N, _D, _H, _C = 16, 1024, 128, 128, 10
_CP = 128


def _fused_body(adj_ref, x_ref, wr_ref, wn_ref, b_ref, wc_ref, bc_ref, out_ref):
    A = (adj_ref[0] > 0.5).astype(jnp.bfloat16)
    xb = x_ref[0]
    agg = jax.lax.dot_general(
        A, xb.astype(jnp.bfloat16),
        dimension_numbers=(((0,), (0,)), ((), ())),
        preferred_element_type=jnp.float32)
    h = jnp.dot(xb, wr_ref[...], preferred_element_type=jnp.float32)
    h = h + jnp.dot(agg, wn_ref[...], preferred_element_type=jnp.float32)
    h = jnp.maximum(h + b_ref[...], 0.0)
    pooled = jnp.sum(h, axis=0, keepdims=True) * (1.0 / _N)
    logits = jnp.dot(pooled, wc_ref[...],
                     preferred_element_type=jnp.float32) + bc_ref[...]
    out_ref[0] = logits


def kernel(x, adj, W_root, W_nbr, b, W_cls, b_cls):
    b2 = b.reshape(1, _H)
    wc = jnp.zeros((_H, _CP), jnp.float32).at[:, :_C].set(W_cls)
    bc = jnp.zeros((1, _CP), jnp.float32).at[0, :_C].set(b_cls)

    out = pl.pallas_call(
        _fused_body,
        grid=(_B,),
        in_specs=[
            pl.BlockSpec((1, _N, _N), lambda i: (i, 0, 0)),
            pl.BlockSpec((1, _N, _D), lambda i: (i, 0, 0)),
            pl.BlockSpec((_D, _H), lambda i: (0, 0)),
            pl.BlockSpec((_D, _H), lambda i: (0, 0)),
            pl.BlockSpec((1, _H), lambda i: (0, 0)),
            pl.BlockSpec((_H, _CP), lambda i: (0, 0)),
            pl.BlockSpec((1, _CP), lambda i: (0, 0)),
        ],
        out_specs=pl.BlockSpec((1, 1, _CP), lambda i: (i, 0, 0)),
        out_shape=jax.ShapeDtypeStruct((_B, 1, _CP), jnp.float32),
    )(adj, x, W_root, W_nbr, b2, wc, bc)
    return out[:, 0, :_C]


# speedup vs baseline: 1.5968x; 1.0009x over previous
import jax
import jax.numpy as jnp
from jax.experimental import pallas as pl
from jax.experimental.pallas import tpu as pltpu

_B, _N, _D, _H, _C = 16, 1024, 128, 128, 10
_PB = 2
_CP = 128


def _fused_body(adj_ref, x_ref, wr_ref, wn_ref, b_ref, wc_ref, bc_ref, out_ref):
    for t in range(_PB):
        A = (adj_ref[t] > 0.5).astype(jnp.bfloat16)
        xb = x_ref[t]
        agg = jax.lax.dot_general(
            A, xb.astype(jnp.bfloat16),
            dimension_numbers=(((0,), (0,)), ((), ())),
            preferred_element_type=jnp.float32)
        h = jnp.dot(xb, wr_ref[...], preferred_element_type=jnp.float32)
        h = h + jnp.dot(agg, wn_ref[...], preferred_element_type=jnp.float32)
        h = jnp.maximum(h + b_ref[...], 0.0)
        pooled = jnp.sum(h, axis=0, keepdims=True) * (1.0 / _N)
        out_ref[t] = jnp.dot(pooled, wc_ref[...],
                             preferred_element_type=jnp.float32) + bc_ref[...]


def kernel(x, adj, W_root, W_nbr, b, W_cls, b_cls):
    b2 = b.reshape(1, _H)
    wc = jnp.zeros((_H, _CP), jnp.float32).at[:, :_C].set(W_cls)
    bc = jnp.zeros((1, _CP), jnp.float32).at[0, :_C].set(b_cls)

    out = pl.pallas_call(
        _fused_body,
        grid=(_B // _PB,),
        in_specs=[
            pl.BlockSpec((_PB, _N, _N), lambda i: (i, 0, 0)),
            pl.BlockSpec((_PB, _N, _D), lambda i: (i, 0, 0)),
            pl.BlockSpec((_D, _H), lambda i: (0, 0)),
            pl.BlockSpec((_D, _H), lambda i: (0, 0)),
            pl.BlockSpec((1, _H), lambda i: (0, 0)),
            pl.BlockSpec((_H, _CP), lambda i: (0, 0)),
            pl.BlockSpec((1, _CP), lambda i: (0, 0)),
        ],
        out_specs=pl.BlockSpec((_PB, 1, _CP), lambda i: (i, 0, 0)),
        out_shape=jax.ShapeDtypeStruct((_B, 1, _CP), jnp.float32),
    )(adj, x, W_root, W_nbr, b2, wc, bc)
    return out[:, 0, :_C]
